# baseline (speedup 1.0000x reference)
import functools

import jax
import jax.numpy as jnp
from jax import lax
from jax.experimental import pallas as pl
from jax.experimental.pallas import tpu as pltpu
from jax.experimental.pallas import tpu_sc as plsc

_LANES = 16
_D = 64
_ZSTRIDE = 65
_GT = 64


def _prep_body(nt_ref, wd_ref, wont_ref, bon_ref, wdut_ref, bdu_ref, bd_ref,
               pa_ref, g4_ref, b4_ref, ftab_ref, par_ref):
    wd = wd_ref[...]
    w0 = wd[0:16, :]
    w1 = wd[16:24, :]
    w2 = wd[24:28, :]
    c = bon_ref[...] @ w1 + bdu_ref[...] @ w2 + bd_ref[...]
    t = jnp.dot(nt_ref[...], w0, preferred_element_type=jnp.float32)
    ftab_ref[...] = t + c
    ut = lax.dot_general(w1, wont_ref[...], (((0,), (0,)), ((), ())))
    vt = lax.dot_general(w2, wdut_ref[...], (((0,), (0,)), ((), ())))
    dims = (_D, _LANES)
    par_ref[...] = jnp.concatenate([
        jnp.broadcast_to(ut, dims),
        jnp.broadcast_to(vt, dims),
        jnp.broadcast_to(g4_ref[...], dims),
        jnp.broadcast_to(b4_ref[...], dims),
        jnp.broadcast_to(pa_ref[...], (8, _LANES)),
    ], axis=0)


def _prep(note_table, W_on, b_on, W_du, b_du, W_dense, b_dense,
          prelu_a, gamma, beta):
    nt = jnp.zeros((96, 16), jnp.float32).at[:91, :].set(note_table)
    return pl.pallas_call(
        _prep_body,
        out_shape=[
            jax.ShapeDtypeStruct((96, _D), jnp.float32),
            jax.ShapeDtypeStruct((264, _LANES), jnp.float32),
        ],
    )(nt, W_dense, W_on.reshape(8, 1), b_on.reshape(1, 8),
      W_du.reshape(4, 1), b_du.reshape(1, 4), b_dense.reshape(1, _D),
      prelu_a.reshape(1, 1), gamma.reshape(_D, 1),
      beta.reshape(_D, 1))


def _sc_body(num_cores, tok_per_tile, chunk, nchunks,
             ftab_hbm, par_hbm, notes_hbm, on_hbm, du_hbm, out_hbm,
             tab_v, par_v,
             notes0_v, on0_v, du0_v, outbuf0_v, sem_in0, sem_out0,
             notes1_v, on1_v, du1_v, outbuf1_v, sem_in1, sem_out1,
             zbuf_v):
    wid = lax.axis_index("s") * num_cores + lax.axis_index("c")
    tile_base = wid * tok_per_tile
    pltpu.sync_copy(ftab_hbm, tab_v)
    pltpu.sync_copy(par_hbm, par_v)
    a_vec = par_v[pl.ds(4 * _D * _LANES, _LANES)]
    lane = lax.iota(jnp.int32, _LANES)
    zbase = [(lane + q * _LANES) * _ZSTRIDE for q in range(4)]
    f32z = jnp.zeros((_LANES,), jnp.float32)

    bufs = ((notes0_v, on0_v, du0_v, outbuf0_v, sem_in0, sem_out0),
            (notes1_v, on1_v, du1_v, outbuf1_v, sem_in1, sem_out1))

    def issue_in(base, p):
        nv, ov, dv_, _, si, _ = bufs[p]
        pltpu.async_copy(notes_hbm.at[pl.ds(base, chunk)], nv, si)
        pltpu.async_copy(on_hbm.at[pl.ds(base, chunk)], ov, si)
        pltpu.async_copy(du_hbm.at[pl.ds(base, chunk)], dv_, si)

    def wait_in(base, p):
        nv, ov, dv_, _, si, _ = bufs[p]
        pltpu.make_async_copy(notes_hbm.at[pl.ds(base, chunk)], nv, si).wait()
        pltpu.make_async_copy(on_hbm.at[pl.ds(base, chunk)], ov, si).wait()
        pltpu.make_async_copy(du_hbm.at[pl.ds(base, chunk)], dv_, si).wait()

    def out_slice(base):
        return out_hbm.at[pl.ds(base * _D, chunk * _D)]

    def do_chunk(ci, p):
        base = tile_base + ci * chunk
        notes_v, on_v, du_v, outbuf_v, _, so = bufs[p]
        wait_in(base, p)
        pltpu.make_async_copy(outbuf_v, out_slice(base), so).wait()

        def group_body(g, carry2):
            off = g * _GT
            gidx = [notes_v[pl.ds(off + q * _LANES, _LANES)] * (_D * _LANES)
                    + lane for q in range(4)]
            onv = [on_v[pl.ds(off + q * _LANES, _LANES)] for q in range(4)]
            duv = [du_v[pl.ds(off + q * _LANES, _LANES)] for q in range(4)]

            @plsc.parallel_loop(0, _D, 1, unroll=4, carry=(f32z,) * 8)
            def acc(d, acc_in):
                s1 = list(acc_in[:4])
                s2 = list(acc_in[4:])
                d16 = jnp.full((_LANES,), 0, jnp.int32) + d * _LANES
                dv = jnp.full((_LANES,), 0, jnp.int32) + d
                ud = par_v[pl.ds(d * _LANES, _LANES)]
                vd = par_v[pl.ds(_D * _LANES + d * _LANES, _LANES)]
                for q in range(4):
                    r = plsc.load_gather(tab_v, [gidx[q] + d16])
                    z = r + onv[q] * ud + duv[q] * vd
                    z = jnp.where(z >= 0.0, z, z * a_vec)
                    s1[q] = s1[q] + z
                    s2[q] = s2[q] + z * z
                    plsc.store_scatter(zbuf_v, [zbase[q] + dv], z)
                return tuple(s1) + tuple(s2)

            ys, b0s = [], []
            for q in range(4):
                mu = acc[q] * (1.0 / _D)
                var = acc[4 + q] * (1.0 / _D) - mu * mu
                x = var + 1e-5
                i = plsc.bitcast(x, jnp.int32)
                i = jnp.int32(0x5F3759DF) - lax.shift_right_logical(i, 1)
                y = plsc.bitcast(i, jnp.float32)
                y = y * (1.5 - 0.5 * x * y * y)
                y = y * (1.5 - 0.5 * x * y * y)
                ys.append(y)
                b0s.append(-(mu * y))

            @plsc.parallel_loop(0, _D, 1, unroll=4)
            def _(d):
                dv = jnp.full((_LANES,), 0, jnp.int32) + d
                gd = par_v[pl.ds(2 * _D * _LANES + d * _LANES, _LANES)]
                bd = par_v[pl.ds(3 * _D * _LANES + d * _LANES, _LANES)]
                for q in range(4):
                    idx = zbase[q] + dv
                    z = plsc.load_gather(zbuf_v, [idx])
                    o = (z * ys[q] + b0s[q]) * gd + bd
                    plsc.store_scatter(zbuf_v, [idx], o)

            off64 = off * _D

            @plsc.parallel_loop(0, _GT, 1, unroll=8)
            def _(j):
                zrow = j * _ZSTRIDE
                orow = off64 + j * _D
                for q2 in range(4):
                    outbuf_v[pl.ds(orow + q2 * _LANES, _LANES)] = (
                        zbuf_v[pl.ds(zrow + q2 * _LANES, _LANES)])
            return carry2

        lax.fori_loop(0, chunk // _GT, group_body, 0)
        pltpu.async_copy(outbuf_v, out_slice(base), so)
        nxt = jnp.minimum(ci + 2, nchunks - 2 + p)
        issue_in(tile_base + nxt * chunk, p)

    issue_in(tile_base, 0)
    issue_in(tile_base + chunk, 1)
    pltpu.async_copy(outbuf0_v, out_slice(tile_base), sem_out0)
    pltpu.async_copy(outbuf1_v, out_slice(tile_base + chunk), sem_out1)

    def pair_body(k, carry):
        do_chunk(k * 2, 0)
        do_chunk(k * 2 + 1, 1)
        return carry

    lax.fori_loop(0, nchunks // 2, pair_body, 0)
    last = tile_base + (nchunks - 2) * chunk
    wait_in(last, 0)
    wait_in(last + chunk, 1)
    pltpu.make_async_copy(outbuf0_v, out_slice(last), sem_out0).wait()
    pltpu.make_async_copy(outbuf1_v, out_slice(last + chunk), sem_out1).wait()


def kernel(notes, onsets, durations, note_table, W_on, b_on, W_du, b_du,
           W_dense, b_dense, prelu_a, gamma, beta):
    bsz, seq, _ = notes.shape
    n_tok = bsz * seq
    notes_f = jnp.maximum(notes.reshape(n_tok), 0).astype(jnp.int32)
    on_f = jnp.abs(onsets.reshape(n_tok).astype(jnp.float32))
    du_f = jnp.abs(durations.reshape(n_tok).astype(jnp.float32))

    ftab, par = _prep(note_table, W_on, b_on, W_du, b_du, W_dense, b_dense,
                      prelu_a, gamma, beta)
    ftab_rep = jnp.broadcast_to(ftab.reshape(96 * _D, 1),
                                (96 * _D, _LANES)).reshape(96 * _D * _LANES)

    mesh = plsc.VectorSubcoreMesh(core_axis_name="c", subcore_axis_name="s")
    n_tiles = mesh.num_cores * mesh.num_subcores
    tok_per_tile = n_tok // n_tiles
    chunk = 128
    nchunks = tok_per_tile // chunk

    body = functools.partial(_sc_body, mesh.num_cores, tok_per_tile,
                             chunk, nchunks)
    out = pl.kernel(
        body,
        out_type=jax.ShapeDtypeStruct((n_tok * _D,), jnp.float32),
        mesh=mesh,
        compiler_params=pltpu.CompilerParams(needs_layout_passes=False),
        scratch_types=(
            [pltpu.VMEM((96 * _D * _LANES,), jnp.float32),
             pltpu.VMEM((264 * _LANES,), jnp.float32)]
            + 2 * [pltpu.VMEM((chunk,), jnp.int32),
                   pltpu.VMEM((chunk,), jnp.float32),
                   pltpu.VMEM((chunk,), jnp.float32),
                   pltpu.VMEM((chunk * _D,), jnp.float32),
                   pltpu.SemaphoreType.DMA,
                   pltpu.SemaphoreType.DMA]
            + [pltpu.VMEM((_GT * _ZSTRIDE,), jnp.float32)]
        ),
    )(ftab_rep, par.reshape(264 * _LANES), notes_f, on_f, du_f)
    return out.reshape(bsz, seq, _D)


# speedup vs baseline: 1.0248x; 1.0248x over previous
import functools

import jax
import jax.numpy as jnp
from jax import lax
from jax.experimental import pallas as pl
from jax.experimental.pallas import tpu as pltpu
from jax.experimental.pallas import tpu_sc as plsc

_LANES = 16
_D = 64
_ZSTRIDE = 65
_GT = 64


def _prep_body(nt_ref, wd_ref, wont_ref, bon_ref, wdut_ref, bdu_ref, bd_ref,
               pa_ref, g4_ref, b4_ref, ftab_ref, par_ref):
    wd = wd_ref[...]
    w0 = wd[0:16, :]
    w1 = wd[16:24, :]
    w2 = wd[24:28, :]
    c = bon_ref[...] @ w1 + bdu_ref[...] @ w2 + bd_ref[...]
    t = jnp.dot(nt_ref[...], w0, preferred_element_type=jnp.float32)
    ftab_ref[...] = t + c
    ut = lax.dot_general(w1, wont_ref[...], (((0,), (0,)), ((), ())))
    vt = lax.dot_general(w2, wdut_ref[...], (((0,), (0,)), ((), ())))
    dims = (_D, _LANES)
    par_ref[...] = jnp.concatenate([
        jnp.broadcast_to(ut, dims),
        jnp.broadcast_to(vt, dims),
        jnp.broadcast_to(g4_ref[...], dims),
        jnp.broadcast_to(b4_ref[...], dims),
        jnp.broadcast_to(pa_ref[...], (8, _LANES)),
    ], axis=0)


def _prep(note_table, W_on, b_on, W_du, b_du, W_dense, b_dense,
          prelu_a, gamma, beta):
    nt = jnp.zeros((96, 16), jnp.float32).at[:91, :].set(note_table)
    return pl.pallas_call(
        _prep_body,
        out_shape=[
            jax.ShapeDtypeStruct((96, _D), jnp.float32),
            jax.ShapeDtypeStruct((264, _LANES), jnp.float32),
        ],
    )(nt, W_dense, W_on.reshape(8, 1), b_on.reshape(1, 8),
      W_du.reshape(4, 1), b_du.reshape(1, 4), b_dense.reshape(1, _D),
      prelu_a.reshape(1, 1), gamma.reshape(_D, 1),
      beta.reshape(_D, 1))


def _sc_body(num_cores, tok_per_tile, chunk, nchunks,
             ftab_hbm, par_hbm, notes_hbm, on_hbm, du_hbm, out_hbm,
             tab_v, par_v,
             notes0_v, on0_v, du0_v, outbuf0_v, sem_in0, sem_out0,
             notes1_v, on1_v, du1_v, outbuf1_v, sem_in1, sem_out1,
             zbuf_v):
    wid = lax.axis_index("s") * num_cores + lax.axis_index("c")
    tile_base = wid * tok_per_tile
    pltpu.sync_copy(ftab_hbm, tab_v)
    pltpu.sync_copy(par_hbm, par_v)
    a_vec = par_v[pl.ds(4 * _D * _LANES, _LANES)]
    lane = lax.iota(jnp.int32, _LANES)
    f32z = jnp.zeros((_LANES,), jnp.float32)

    zbase = [(lane + q * _LANES) * _ZSTRIDE for q in range(4)]
    bufs = ((notes0_v, on0_v, du0_v, outbuf0_v, sem_in0, sem_out0),
            (notes1_v, on1_v, du1_v, outbuf1_v, sem_in1, sem_out1))

    def issue_in(base, p):
        nv, ov, dv_, _, si, _ = bufs[p]
        pltpu.async_copy(notes_hbm.at[pl.ds(base, chunk)], nv, si)
        pltpu.async_copy(on_hbm.at[pl.ds(base, chunk)], ov, si)
        pltpu.async_copy(du_hbm.at[pl.ds(base, chunk)], dv_, si)

    def wait_in(base, p):
        nv, ov, dv_, _, si, _ = bufs[p]
        pltpu.make_async_copy(notes_hbm.at[pl.ds(base, chunk)], nv, si).wait()
        pltpu.make_async_copy(on_hbm.at[pl.ds(base, chunk)], ov, si).wait()
        pltpu.make_async_copy(du_hbm.at[pl.ds(base, chunk)], dv_, si).wait()

    def out_slice(base):
        return out_hbm.at[pl.ds(base * _D, chunk * _D)]

    def do_chunk(ci, p):
        base = tile_base + ci * chunk
        notes_v, on_v, du_v, outbuf_v, _, so = bufs[p]
        wait_in(base, p)
        pltpu.make_async_copy(outbuf_v, out_slice(base), so).wait()

        def group_body(g, carry2):
            off = g * _GT
            gidx = [notes_v[pl.ds(off + q * _LANES, _LANES)] * (_D * _LANES)
                    + lane for q in range(4)]
            onv = [on_v[pl.ds(off + q * _LANES, _LANES)] for q in range(4)]
            duv = [du_v[pl.ds(off + q * _LANES, _LANES)] for q in range(4)]

            @plsc.parallel_loop(0, _D, 1, unroll=4, carry=(f32z,) * 8)
            def acc(d, acc_in):
                s1 = list(acc_in[:4])
                s2 = list(acc_in[4:])
                d16 = jnp.full((_LANES,), 0, jnp.int32) + d * _LANES
                dv = jnp.full((_LANES,), 0, jnp.int32) + d
                ud = par_v[pl.ds(d * _LANES, _LANES)]
                vd = par_v[pl.ds(_D * _LANES + d * _LANES, _LANES)]
                for q in range(4):
                    r = plsc.load_gather(tab_v, [gidx[q] + d16])
                    z = r + onv[q] * ud + duv[q] * vd
                    z = jnp.where(z >= 0.0, z, z * a_vec)
                    s1[q] = s1[q] + z
                    s2[q] = s2[q] + z * z
                    plsc.store_scatter(zbuf_v, [zbase[q] + dv], z)
                return tuple(s1) + tuple(s2)

            ys, b0s = [], []
            for q in range(4):
                mu = acc[q] * (1.0 / _D)
                var = acc[4 + q] * (1.0 / _D) - mu * mu
                x = var + 1e-5
                i = plsc.bitcast(x, jnp.int32)
                i = jnp.int32(0x5F3759DF) - lax.shift_right_logical(i, 1)
                y = plsc.bitcast(i, jnp.float32)
                y = y * (1.5 - 0.5 * x * y * y)
                y = y * (1.5 - 0.5 * x * y * y)
                ys.append(y)
                b0s.append(-(mu * y))

            @plsc.parallel_loop(0, _D, 1, unroll=4)
            def _(d):
                dv = jnp.full((_LANES,), 0, jnp.int32) + d
                gd = par_v[pl.ds(2 * _D * _LANES + d * _LANES, _LANES)]
                bd = par_v[pl.ds(3 * _D * _LANES + d * _LANES, _LANES)]
                for q in range(4):
                    idx = zbase[q] + dv
                    z = plsc.load_gather(zbuf_v, [idx])
                    o = (z * ys[q] + b0s[q]) * gd + bd
                    plsc.store_scatter(zbuf_v, [idx], o)

            off64 = off * _D

            @plsc.parallel_loop(0, _GT, 1, unroll=8)
            def _(j):
                zrow = j * _ZSTRIDE
                orow = off64 + j * _D
                for q2 in range(4):
                    outbuf_v[pl.ds(orow + q2 * _LANES, _LANES)] = (
                        zbuf_v[pl.ds(zrow + q2 * _LANES, _LANES)])
            return carry2

        lax.fori_loop(0, chunk // _GT, group_body, 0)
        pltpu.async_copy(outbuf_v, out_slice(base), so)
        nxt = jnp.minimum(ci + 2, nchunks - 2 + p)
        issue_in(tile_base + nxt * chunk, p)

    issue_in(tile_base, 0)
    issue_in(tile_base + chunk, 1)
    pltpu.async_copy(outbuf0_v, out_slice(tile_base), sem_out0)
    pltpu.async_copy(outbuf1_v, out_slice(tile_base + chunk), sem_out1)

    def pair_body(k, carry):
        do_chunk(k * 2, 0)
        do_chunk(k * 2 + 1, 1)
        return carry

    lax.fori_loop(0, nchunks // 2, pair_body, 0)
    last = tile_base + (nchunks - 2) * chunk
    wait_in(last, 0)
    wait_in(last + chunk, 1)
    pltpu.make_async_copy(outbuf0_v, out_slice(last), sem_out0).wait()
    pltpu.make_async_copy(outbuf1_v, out_slice(last + chunk), sem_out1).wait()


def kernel(notes, onsets, durations, note_table, W_on, b_on, W_du, b_du,
           W_dense, b_dense, prelu_a, gamma, beta):
    bsz, seq, _ = notes.shape
    n_tok = bsz * seq
    notes_f = notes.reshape(n_tok).astype(jnp.int32)
    on_f = onsets.reshape(n_tok).astype(jnp.float32)
    du_f = durations.reshape(n_tok).astype(jnp.float32)

    ftab, par = _prep(note_table, W_on, b_on, W_du, b_du, W_dense, b_dense,
                      prelu_a, gamma, beta)
    ftab_rep = jnp.broadcast_to(ftab.reshape(96 * _D, 1),
                                (96 * _D, _LANES)).reshape(96 * _D * _LANES)

    mesh = plsc.VectorSubcoreMesh(core_axis_name="c", subcore_axis_name="s")
    n_tiles = mesh.num_cores * mesh.num_subcores
    tok_per_tile = n_tok // n_tiles
    chunk = 128
    nchunks = tok_per_tile // chunk

    body = functools.partial(_sc_body, mesh.num_cores, tok_per_tile,
                             chunk, nchunks)
    out = pl.kernel(
        body,
        out_type=jax.ShapeDtypeStruct((n_tok * _D,), jnp.float32),
        mesh=mesh,
        compiler_params=pltpu.CompilerParams(needs_layout_passes=False),
        scratch_types=(
            [pltpu.VMEM((96 * _D * _LANES,), jnp.float32),
             pltpu.VMEM((264 * _LANES,), jnp.float32)]
            + 2 * [pltpu.VMEM((chunk,), jnp.int32),
                   pltpu.VMEM((chunk,), jnp.float32),
                   pltpu.VMEM((chunk,), jnp.float32),
                   pltpu.VMEM((chunk * _D,), jnp.float32),
                   pltpu.SemaphoreType.DMA,
                   pltpu.SemaphoreType.DMA]
            + [pltpu.VMEM((_GT * _ZSTRIDE,), jnp.float32)]
        ),
    )(ftab_rep, par.reshape(264 * _LANES), notes_f, on_f, du_f)
    return out.reshape(bsz, seq, _D)
